# baseline (speedup 1.0000x reference)
import functools

import jax
import jax.numpy as jnp
from jax import lax
from jax.experimental import pallas as pl
from jax.experimental.pallas import tpu as pltpu
from jax.experimental.pallas import tpu_sc as plsc
from jax._src.pallas import core as pallas_core
from jax._src.pallas import mpmd

VOCAB = 100000
DIM = 768
B = 256
H = 14
W = 14
HW = H * W
N = B * HW

NC = 2
NS = 16
NW = NC * NS
PER_SC = N // NC
PER_W = N // NW
CHUNK = 32
NCHUNK = PER_W // CHUNK
SLAB = NS * CHUNK
NSLOT = 3

_VMESH = plsc.VectorSubcoreMesh(core_axis_name="c", subcore_axis_name="s")
_SMESH = plsc.ScalarSubcoreMesh(axis_name="c", num_cores=NC)


def _tec_fn(table, idx, out, idx_v, b0, b1, s0, s1, spm, fill, dsem, credit):
    del out, dsem
    k = lax.axis_index("c")
    s = lax.axis_index("s")
    pltpu.sync_copy(idx.at[k, s], idx_v)

    def fire(c, buf, sem):
        pltpu.async_copy(
            table.at[idx_v.at[pl.ds(c * CHUNK, CHUNK)]], buf, sem
        )

    def drain(c, buf, sem, backpressure=True):
        pltpu.make_async_copy(
            table.at[idx_v.at[pl.ds(c * CHUNK, CHUNK)]], buf, sem
        ).wait()
        if backpressure:
            pl.semaphore_wait(credit, 1)
        pltpu.sync_copy(buf, spm.at[c % NSLOT, pl.ds(s * CHUNK, CHUNK)])
        pltpu.semaphore_signal(fill.at[s])

    fire(0, b0, s0)
    fire(1, b1, s1)

    def body(j, carry):
        c = 2 * j
        drain(c, b0, s0)
        fire(c + 2, b0, s0)
        drain(c + 1, b1, s1)
        fire(c + 3, b1, s1)
        return carry

    drain(0, b0, s0, backpressure=False)
    fire(2, b0, s0)
    drain(1, b1, s1, backpressure=False)
    fire(3, b1, s1)
    drain(2, b0, s0, backpressure=False)
    fire(4, b0, s0)
    drain(3, b1, s1)
    fire(5, b1, s1)
    lax.fori_loop(2, (NCHUNK - 3) // 2, body, 0)
    drain(NCHUNK - 3, b0, s0)
    fire(NCHUNK - 1, b0, s0)
    drain(NCHUNK - 2, b1, s1)
    drain(NCHUNK - 1, b0, s0)


def _scs_fn(table, idx, out, idx_v, b0, b1, s0, s1, spm, fill, dsem, credit):
    del table, idx, idx_v, b0, b1, s0, s1
    k = lax.axis_index("c")

    def round_body(c, carry):
        def fwait(t, _):
            pl.semaphore_wait(fill.at[t], 1)
            return _

        lax.fori_loop(0, NS, fwait, 0)
        base = k * PER_SC + c * SLAB
        pltpu.async_copy(
            spm.at[c % NSLOT], out.at[pl.ds(base, SLAB)], dsem
        ).wait()

        def csig(t, _):
            pltpu.semaphore_signal(credit, 1, device_id={"s": t})
            return _

        lax.fori_loop(0, NS, csig, 0)
        return carry

    lax.fori_loop(0, NCHUNK, round_body, 0)


@jax.jit
def _sc_gather(token_emb, ids):
    return mpmd.mpmd_map(
        [(_SMESH, _scs_fn), (_VMESH, _tec_fn)],
        out_types=[jax.ShapeDtypeStruct((N, DIM), jnp.float32)],
        scratch_types=[
            pallas_core.CoreMemorySpace(pltpu.VMEM, _VMESH)((PER_W,), jnp.int32),
            pallas_core.CoreMemorySpace(pltpu.VMEM, _VMESH)((CHUNK, DIM), jnp.float32),
            pallas_core.CoreMemorySpace(pltpu.VMEM, _VMESH)((CHUNK, DIM), jnp.float32),
            pltpu.SemaphoreType.DMA @ _VMESH,
            pltpu.SemaphoreType.DMA @ _VMESH,
            pltpu.VMEM_SHARED((NSLOT, SLAB, DIM), jnp.float32),
            pallas_core.CoreMemorySpace(pltpu.SEMAPHORE, _SMESH)(
                (NS,), pltpu.SemaphoreType.REGULAR.dtype
            ),
            pltpu.SemaphoreType.DMA @ _SMESH,
            pltpu.SemaphoreType.REGULAR @ _VMESH,
        ],
    )(token_emb, ids)[0]


def _emb_body(pos_ref, mod_ref, out_ref):
    out_ref[...] = jnp.broadcast_to(
        pos_ref[0][:, None, :] + mod_ref[...], out_ref.shape
    )


def _build_2d_sincos_posemb(h, w, embed_dim, temperature=10000.0):
    grid_w = jnp.arange(w, dtype=jnp.float32)
    grid_h = jnp.arange(h, dtype=jnp.float32)
    grid_w, grid_h = jnp.meshgrid(grid_w, grid_h, indexing='ij')
    pos_dim = embed_dim // 4
    omega = jnp.arange(pos_dim, dtype=jnp.float32) / pos_dim
    omega = 1.0 / (temperature ** omega)
    out_w = jnp.einsum('m,d->md', grid_w.flatten(), omega)
    out_h = jnp.einsum('m,d->md', grid_h.flatten(), omega)
    return jnp.concatenate(
        [jnp.sin(out_w), jnp.cos(out_w), jnp.sin(out_h), jnp.cos(out_h)],
        axis=1,
    )


_EMB_BP = 14


@jax.jit
def _tc_emb(pos, mod):
    return pl.pallas_call(
        _emb_body,
        grid=(HW // _EMB_BP,),
        in_specs=[
            pl.BlockSpec((1, _EMB_BP, DIM), lambda i: (i, 0, 0)),
            pl.BlockSpec((1, 1, DIM), lambda i: (0, 0, 0)),
        ],
        out_specs=pl.BlockSpec((_EMB_BP, B, DIM), lambda i: (i, 0, 0)),
        out_shape=jax.ShapeDtypeStruct((HW, B, DIM), jnp.float32),
    )(pos.reshape(HW // _EMB_BP, _EMB_BP, DIM), mod)


def kernel(tensor, token_emb, mod_emb):
    ids_t = tensor.reshape(B, HW).astype(jnp.int32).T.reshape(N)
    ids_r = (
        ids_t.reshape(NC, NCHUNK, NS, CHUNK)
        .transpose(0, 2, 1, 3)
        .reshape(NC, NS, PER_W)
    )
    x_flat = _sc_gather(token_emb, ids_r)
    pos = _build_2d_sincos_posemb(H, W, DIM)
    emb_t = _tc_emb(pos, mod_emb)
    x = jnp.transpose(x_flat.reshape(HW, B, DIM), (1, 0, 2))
    emb = jnp.transpose(emb_t, (1, 0, 2))
    return (x, emb)


# speedup vs baseline: 1.0811x; 1.0811x over previous
import functools

import jax
import jax.numpy as jnp
from jax import lax
from jax.experimental import pallas as pl
from jax.experimental.pallas import tpu as pltpu
from jax.experimental.pallas import tpu_sc as plsc

VOCAB = 100000
DIM = 768
B = 256
H = 14
W = 14
HW = H * W
N = B * HW

NC = 2
NS = 16
NW = NC * NS
PER_W = N // NW
CHUNK = 56
NCHUNK = PER_W // CHUNK


def _sc_gather_body(table_hbm, idx_hbm, out_hbm, idx_v, rows0, rows1, s0, s1):
    wid = lax.axis_index("s") * NC + lax.axis_index("c")
    base = wid * PER_W
    pltpu.sync_copy(idx_hbm.at[pl.ds(base, PER_W)], idx_v)

    def fire(c, buf, sem):
        pltpu.async_copy(
            table_hbm.at[idx_v.at[pl.ds(c * CHUNK, CHUNK)]], buf, sem
        )

    def drain_write(c, buf, sem):
        pltpu.make_async_copy(
            table_hbm.at[idx_v.at[pl.ds(c * CHUNK, CHUNK)]], buf, sem
        ).wait()
        pltpu.sync_copy(buf, out_hbm.at[pl.ds(base + c * CHUNK, CHUNK)])

    fire(0, rows0, s0)
    fire(1, rows1, s1)

    def body(j, carry):
        c = 2 * j
        drain_write(c, rows0, s0)
        fire(c + 2, rows0, s0)
        drain_write(c + 1, rows1, s1)
        fire(c + 3, rows1, s1)
        return carry

    lax.fori_loop(0, NCHUNK // 2 - 1, body, 0)
    drain_write(NCHUNK - 2, rows0, s0)
    drain_write(NCHUNK - 1, rows1, s1)


@jax.jit
def _sc_gather(token_emb, ids):
    mesh = plsc.VectorSubcoreMesh(core_axis_name="c", subcore_axis_name="s")
    return pl.kernel(
        _sc_gather_body,
        out_type=jax.ShapeDtypeStruct((N, DIM), jnp.float32),
        mesh=mesh,
        scratch_types=[
            pltpu.VMEM((PER_W,), jnp.int32),
            pltpu.VMEM((CHUNK, DIM), jnp.float32),
            pltpu.VMEM((CHUNK, DIM), jnp.float32),
            pltpu.SemaphoreType.DMA,
            pltpu.SemaphoreType.DMA,
        ],
    )(token_emb, ids)


def _emb_body(pos_ref, mod_ref, out_ref):
    out_ref[...] = jnp.broadcast_to(
        pos_ref[0][:, None, :] + mod_ref[...], out_ref.shape
    )


def _build_2d_sincos_posemb(h, w, embed_dim, temperature=10000.0):
    grid_w = jnp.arange(w, dtype=jnp.float32)
    grid_h = jnp.arange(h, dtype=jnp.float32)
    grid_w, grid_h = jnp.meshgrid(grid_w, grid_h, indexing='ij')
    pos_dim = embed_dim // 4
    omega = jnp.arange(pos_dim, dtype=jnp.float32) / pos_dim
    omega = 1.0 / (temperature ** omega)
    out_w = jnp.einsum('m,d->md', grid_w.flatten(), omega)
    out_h = jnp.einsum('m,d->md', grid_h.flatten(), omega)
    return jnp.concatenate(
        [jnp.sin(out_w), jnp.cos(out_w), jnp.sin(out_h), jnp.cos(out_h)],
        axis=1,
    )


_EMB_BP = 14


@jax.jit
def _tc_emb(pos, mod):
    return pl.pallas_call(
        _emb_body,
        grid=(HW // _EMB_BP,),
        in_specs=[
            pl.BlockSpec((1, _EMB_BP, DIM), lambda i: (i, 0, 0)),
            pl.BlockSpec((1, 1, DIM), lambda i: (0, 0, 0)),
        ],
        out_specs=pl.BlockSpec((_EMB_BP, B, DIM), lambda i: (i, 0, 0)),
        out_shape=jax.ShapeDtypeStruct((HW, B, DIM), jnp.float32),
    )(pos.reshape(HW // _EMB_BP, _EMB_BP, DIM), mod)


def kernel(tensor, token_emb, mod_emb):
    ids_t = tensor.reshape(B, HW).astype(jnp.int32).T.reshape(N)
    x_flat = _sc_gather(token_emb, ids_t)
    pos = _build_2d_sincos_posemb(H, W, DIM)
    emb_t = _tc_emb(pos, mod_emb)
    x = jnp.transpose(x_flat.reshape(HW, B, DIM), (1, 0, 2))
    emb = jnp.transpose(emb_t, (1, 0, 2))
    return (x, emb)
